# baseline (speedup 1.0000x reference)
import functools
import math

import jax
import jax.numpy as jnp
from jax import lax
from jax.experimental import pallas as pl
from jax.experimental.pallas import tpu as pltpu
from jax.experimental.pallas import tpu_sc as plsc

BINSIZE = 200
BINWIDTH = 500
N_CLUSTERS = 16
N_REGIONS_OI = 256
N_CELLS = 10000
LOG_BINSIZE = math.log(float(BINSIZE))

NC = 2
NS = 16
LANES = 16
NW = NC * NS

CHUNK = 128
CPW = 124
UNROLL = 4
BPW = CHUNK * CPW
NPAD = BPW * NW
N_CELLS_PAD = 10240


RPB = 8


def _heights_body(roi_ref, *refs):
    base_refs = refs[:RPB]
    delta_refs = refs[RPB:2 * RPB]
    out_ref = refs[2 * RPB]
    for k in range(RPB):
        x = base_refs[k][0] + delta_refs[k][0]
        m = jnp.max(x, axis=-1, keepdims=True)
        lse = jnp.log(jnp.sum(jnp.exp(x - m), axis=-1, keepdims=True)) + m
        out_ref[k] = (x - lse - LOG_BINSIZE).astype(jnp.bfloat16)


def _compute_heights(baseline_weight, delta_logit_weight, regions_oi):
    baseline3 = baseline_weight.reshape(baseline_weight.shape[0], 1, BINWIDTH)

    def base_map(k):
        return lambda i, roi: (roi[i * RPB + k], 0, 0)

    grid_spec = pltpu.PrefetchScalarGridSpec(
        num_scalar_prefetch=1,
        grid=(N_REGIONS_OI // RPB,),
        in_specs=(
            [pl.BlockSpec((1, 1, BINWIDTH), base_map(k)) for k in range(RPB)]
            + [pl.BlockSpec((1, N_CLUSTERS, BINWIDTH), base_map(k)) for k in range(RPB)]
        ),
        out_specs=pl.BlockSpec((RPB, N_CLUSTERS, BINWIDTH), lambda i, roi: (i, 0, 0)),
    )
    args = [baseline3] * RPB + [delta_logit_weight] * RPB
    return pl.pallas_call(
        _heights_body,
        grid_spec=grid_spec,
        out_shape=jax.ShapeDtypeStruct((N_REGIONS_OI, N_CLUSTERS, BINWIDTH), jnp.bfloat16),
    )(regions_oi, *args)


HWORDS = N_REGIONS_OI * N_CLUSTERS * BINWIDTH // 2


@functools.lru_cache(maxsize=1)
def _make_gather_kernel():
    mesh = plsc.VectorSubcoreMesh(core_axis_name="c", subcore_axis_name="s")
    HPT = HWORDS // NS

    @functools.partial(
        pl.kernel,
        mesh=mesh,
        out_type=jax.ShapeDtypeStruct((NW, CPW, CHUNK), jnp.int32),
        scratch_types=[
            pltpu.VMEM_SHARED((N_CELLS_PAD,), jnp.int32),
            pltpu.VMEM((CPW, CHUNK), jnp.int32),
            pltpu.VMEM((CPW, CHUNK), jnp.int32),
            pltpu.VMEM((CPW, CHUNK), jnp.int32),
            pltpu.VMEM((CPW, CHUNK), jnp.int32),
            pltpu.VMEM((CPW, CHUNK), jnp.int32),
            pltpu.VMEM((CPW, CHUNK), jnp.int32),
            pltpu.VMEM((CPW, CHUNK), jnp.int32),
            pltpu.SemaphoreType.DMA,
        ],
    )
    def sc_gather(heights_hbm, labels_hbm, cell_hbm, reg_hbm, coord_hbm, out_hbm,
                  labels_v, cell_v, reg_v, coord_v, cluster_v, flat_v,
                  word_v, vals_v, sem):
        sid = lax.axis_index("s")
        wid = sid * NC + lax.axis_index("c")

        with jax.named_scope("ph_in"):
            lbl_slice = pl.ds(sid * (N_CELLS_PAD // NS), N_CELLS_PAD // NS)
            pltpu.sync_copy(labels_hbm.at[lbl_slice], labels_v.at[lbl_slice])
            pltpu.sync_copy(cell_hbm.at[wid], cell_v)
            pltpu.sync_copy(reg_hbm.at[wid], reg_v)
            pltpu.sync_copy(coord_hbm.at[wid], coord_v)
            plsc.subcore_barrier()

        with jax.named_scope("ph_lbl"):
            def lbl_fire(g, carry):
                for u in range(UNROLL):
                    j = g * UNROLL + u
                    pltpu.async_copy(labels_v.at[cell_v.at[j]], cluster_v.at[j], sem)
                return carry

            lax.fori_loop(0, CPW // UNROLL, lbl_fire, 0)

            def lbl_drain(g, carry):
                for u in range(UNROLL):
                    j = g * UNROLL + u
                    pltpu.make_async_copy(
                        labels_v.at[cell_v.at[j]], cluster_v.at[j], sem
                    ).wait()
                return carry

            lax.fori_loop(0, CPW // UNROLL, lbl_drain, 0)

        with jax.named_scope("ph_idx"):
            def idx_body(j, carry):
                for k in range(CHUNK // LANES):
                    sl = pl.ds(k * LANES, LANES)
                    cluster = cluster_v[j, sl]
                    reg = reg_v[j, sl]
                    binix = ((coord_v[j, sl] >> 3) * 41944) >> 20
                    flat = (
                        reg * (N_CLUSTERS * BINWIDTH) + cluster * BINWIDTH + binix
                    )
                    flat_v[j, sl] = flat
                    word_v[j, sl] = flat >> 1
                return carry

            lax.fori_loop(0, CPW, idx_body, 0)

        with jax.named_scope("ph_hgt"):
            def hgt_fire(g, carry):
                for u in range(UNROLL):
                    j = g * UNROLL + u
                    pltpu.async_copy(heights_hbm.at[word_v.at[j]], cluster_v.at[j], sem)
                return carry

            lax.fori_loop(0, CPW // UNROLL, hgt_fire, 0)

            def hgt_drain(g, carry):
                for u in range(UNROLL):
                    j = g * UNROLL + u
                    pltpu.make_async_copy(
                        heights_hbm.at[word_v.at[j]], cluster_v.at[j], sem
                    ).wait()
                return carry

            lax.fori_loop(0, CPW // UNROLL, hgt_drain, 0)

        with jax.named_scope("ph_unpack"):
            def unpack_body(j, carry):
                for k in range(CHUNK // LANES):
                    sl = pl.ds(k * LANES, LANES)
                    w = cluster_v[j, sl]
                    odd = flat_v[j, sl] & 1
                    vals_v[j, sl] = jnp.where(odd == 0, w << 16, w & jnp.int32(-65536))
                return carry

            lax.fori_loop(0, CPW, unpack_body, 0)

        with jax.named_scope("ph_out"):
            pltpu.sync_copy(vals_v, out_hbm.at[wid])

    return sc_gather


def kernel(baseline_weight, delta_logit_weight, regions_oi, coordinates,
           local_region_ix, local_cell_ix, labels):
    n = coordinates.shape[0]
    roi = regions_oi.astype(jnp.int32)
    heights = _compute_heights(baseline_weight, delta_logit_weight, roi)
    heights_flat = jax.lax.bitcast_convert_type(
        heights.reshape(-1).reshape(HWORDS, 2), jnp.int32)

    pad = NPAD - n
    shape3 = (NW, CPW, CHUNK)
    cell = jnp.pad(local_cell_ix.astype(jnp.int32), (0, pad)).reshape(shape3)
    reg = jnp.pad(local_region_ix.astype(jnp.int32), (0, pad)).reshape(shape3)
    coord0 = jnp.pad(coordinates[:, 0].astype(jnp.int32), (0, pad)).reshape(shape3)
    labels32 = jnp.pad(labels.astype(jnp.int32), (0, N_CELLS_PAD - labels.shape[0]))

    vals = _make_gather_kernel()(heights_flat, labels32, cell, reg, coord0)
    vals = jax.lax.bitcast_convert_type(vals.reshape(-1)[:n], jnp.float32)
    return jnp.pad(vals[:, None], ((0, 0), (0, 1)))


# speedup vs baseline: 3.3320x; 3.3320x over previous
import functools
import math

import jax
import jax.numpy as jnp
from jax import lax
from jax.experimental import pallas as pl
from jax.experimental.pallas import tpu as pltpu
from jax.experimental.pallas import tpu_sc as plsc

BINSIZE = 200
BINWIDTH = 500
N_CLUSTERS = 16
N_REGIONS_OI = 256
N_CELLS = 10000
LOG_BINSIZE = math.log(float(BINSIZE))

NC = 2
NS = 16
LANES = 16
NW = NC * NS

CHUNK = 128
CPW = 124
UNROLL = 4
BPW = CHUNK * CPW
NPAD = BPW * NW
N_CELLS_PAD = 10240


RPB = 8


def _heights_body(roi_ref, *refs):
    base_refs = refs[:RPB]
    delta_refs = refs[RPB:2 * RPB]
    out_ref = refs[2 * RPB]
    for k in range(RPB):
        x = base_refs[k][0] + delta_refs[k][0]
        m = jnp.max(x, axis=-1, keepdims=True)
        lse = jnp.log(jnp.sum(jnp.exp(x - m), axis=-1, keepdims=True)) + m
        out_ref[k] = x - lse - LOG_BINSIZE


def _compute_heights(baseline_weight, delta_logit_weight, regions_oi):
    baseline3 = baseline_weight.reshape(baseline_weight.shape[0], 1, BINWIDTH)

    def base_map(k):
        return lambda i, roi: (roi[i * RPB + k], 0, 0)

    grid_spec = pltpu.PrefetchScalarGridSpec(
        num_scalar_prefetch=1,
        grid=(N_REGIONS_OI // RPB,),
        in_specs=(
            [pl.BlockSpec((1, 1, BINWIDTH), base_map(k)) for k in range(RPB)]
            + [pl.BlockSpec((1, N_CLUSTERS, BINWIDTH), base_map(k)) for k in range(RPB)]
        ),
        out_specs=pl.BlockSpec((RPB, N_CLUSTERS, BINWIDTH), lambda i, roi: (i, 0, 0)),
    )
    args = [baseline3] * RPB + [delta_logit_weight] * RPB
    return pl.pallas_call(
        _heights_body,
        grid_spec=grid_spec,
        out_shape=jax.ShapeDtypeStruct((N_REGIONS_OI, N_CLUSTERS, BINWIDTH), jnp.float32),
    )(regions_oi, *args)


HWORDS = N_REGIONS_OI * N_CLUSTERS * BINWIDTH // 2


@functools.lru_cache(maxsize=1)
def _make_gather_kernel():
    mesh = plsc.VectorSubcoreMesh(core_axis_name="c", subcore_axis_name="s")
    HPT = HWORDS // NS

    @functools.partial(
        pl.kernel,
        mesh=mesh,
        out_type=jax.ShapeDtypeStruct((NW, CPW, CHUNK), jnp.float32),
        scratch_types=[
            pltpu.VMEM_SHARED((N_CELLS_PAD,), jnp.int32),
            pltpu.VMEM((CPW, CHUNK), jnp.int32),
            pltpu.VMEM((CPW, CHUNK), jnp.int32),
            pltpu.VMEM((CPW, CHUNK), jnp.int32),
            pltpu.VMEM((CPW, CHUNK), jnp.int32),
            pltpu.VMEM((CPW, CHUNK), jnp.int32),
            pltpu.VMEM((CPW, CHUNK), jnp.float32),
            pltpu.SemaphoreType.DMA,
        ],
    )
    def sc_gather(heights_hbm, labels_hbm, cell_hbm, reg_hbm, coord_hbm, out_hbm,
                  labels_v, cell_v, reg_v, coord_v, cluster_v, flat_v,
                  vals_v, sem):
        sid = lax.axis_index("s")
        wid = sid * NC + lax.axis_index("c")

        with jax.named_scope("ph_in"):
            lbl_slice = pl.ds(sid * (N_CELLS_PAD // NS), N_CELLS_PAD // NS)
            pltpu.sync_copy(labels_hbm.at[lbl_slice], labels_v.at[lbl_slice])
            pltpu.sync_copy(cell_hbm.at[wid], cell_v)
            pltpu.sync_copy(reg_hbm.at[wid], reg_v)
            pltpu.sync_copy(coord_hbm.at[wid], coord_v)
            plsc.subcore_barrier()

        with jax.named_scope("ph_lbl"):
            def lbl_fire(g, carry):
                for u in range(UNROLL):
                    j = g * UNROLL + u
                    pltpu.async_copy(labels_v.at[cell_v.at[j]], cluster_v.at[j], sem)
                return carry

            lax.fori_loop(0, CPW // UNROLL, lbl_fire, 0)

            def lbl_drain(g, carry):
                for u in range(UNROLL):
                    j = g * UNROLL + u
                    pltpu.make_async_copy(
                        labels_v.at[cell_v.at[j]], cluster_v.at[j], sem
                    ).wait()
                return carry

            lax.fori_loop(0, CPW // UNROLL, lbl_drain, 0)

        with jax.named_scope("ph_idx"):
            def idx_body(j, carry):
                for k in range(CHUNK // LANES):
                    sl = pl.ds(k * LANES, LANES)
                    cluster = cluster_v[j, sl]
                    reg = reg_v[j, sl]
                    binix = ((coord_v[j, sl] >> 3) * 41944) >> 20
                    flat_v[j, sl] = (
                        reg * (N_CLUSTERS * BINWIDTH) + cluster * BINWIDTH + binix
                    )
                return carry

            lax.fori_loop(0, CPW, idx_body, 0)

        with jax.named_scope("ph_hgt"):
            def hgt_fire(g, carry):
                for u in range(UNROLL):
                    j = g * UNROLL + u
                    pltpu.async_copy(heights_hbm.at[flat_v.at[j]], vals_v.at[j], sem)
                return carry

            lax.fori_loop(0, CPW // UNROLL, hgt_fire, 0)

            def hgt_drain(g, carry):
                for u in range(UNROLL):
                    j = g * UNROLL + u
                    pltpu.make_async_copy(
                        heights_hbm.at[flat_v.at[j]], vals_v.at[j], sem
                    ).wait()
                return carry

            lax.fori_loop(0, CPW // UNROLL, hgt_drain, 0)

        with jax.named_scope("ph_out"):
            pltpu.sync_copy(vals_v, out_hbm.at[wid])

    return sc_gather


def kernel(baseline_weight, delta_logit_weight, regions_oi, coordinates,
           local_region_ix, local_cell_ix, labels):
    n = coordinates.shape[0]
    roi = regions_oi.astype(jnp.int32)
    heights = _compute_heights(baseline_weight, delta_logit_weight, roi)
    heights_flat = heights.reshape(-1)

    pad = NPAD - n
    shape3 = (NW, CPW, CHUNK)
    cell = jnp.pad(local_cell_ix.astype(jnp.int32), (0, pad)).reshape(shape3)
    reg = jnp.pad(local_region_ix.astype(jnp.int32), (0, pad)).reshape(shape3)
    coord0 = jnp.pad(coordinates[:, 0].astype(jnp.int32), (0, pad)).reshape(shape3)
    labels32 = jnp.pad(labels.astype(jnp.int32), (0, N_CELLS_PAD - labels.shape[0]))

    vals = _make_gather_kernel()(heights_flat, labels32, cell, reg, coord0)
    vals = vals.reshape(-1)[:n]
    return jnp.pad(vals[:, None], ((0, 0), (0, 1)))


# speedup vs baseline: 3.5131x; 1.0543x over previous
import functools
import math

import jax
import jax.numpy as jnp
from jax import lax
from jax.experimental import pallas as pl
from jax.experimental.pallas import tpu as pltpu
from jax.experimental.pallas import tpu_sc as plsc

BINSIZE = 200
BINWIDTH = 500
N_CLUSTERS = 16
N_REGIONS_OI = 256
N_CELLS = 10000
LOG_BINSIZE = math.log(float(BINSIZE))

NC = 2
NS = 16
LANES = 16
NW = NC * NS

CHUNK = 128
CPW = 124
UNROLL = 4
BPW = CHUNK * CPW
NPAD = BPW * NW
N_CELLS_PAD = 10240


RPB = 16


def _heights_body(roi_ref, *refs):
    base_refs = refs[:RPB]
    delta_refs = refs[RPB:2 * RPB]
    out_ref = refs[2 * RPB]
    for k in range(RPB):
        x = base_refs[k][0] + delta_refs[k][0]
        m = jnp.max(x, axis=-1, keepdims=True)
        lse = jnp.log(jnp.sum(jnp.exp(x - m), axis=-1, keepdims=True)) + m
        out_ref[k] = x - lse - LOG_BINSIZE


def _compute_heights(baseline_weight, delta_logit_weight, regions_oi):
    baseline3 = baseline_weight.reshape(baseline_weight.shape[0], 1, BINWIDTH)

    def base_map(k):
        return lambda i, roi: (roi[i * RPB + k], 0, 0)

    grid_spec = pltpu.PrefetchScalarGridSpec(
        num_scalar_prefetch=1,
        grid=(N_REGIONS_OI // RPB,),
        in_specs=(
            [pl.BlockSpec((1, 1, BINWIDTH), base_map(k)) for k in range(RPB)]
            + [pl.BlockSpec((1, N_CLUSTERS, BINWIDTH), base_map(k)) for k in range(RPB)]
        ),
        out_specs=pl.BlockSpec((RPB, N_CLUSTERS, BINWIDTH), lambda i, roi: (i, 0, 0)),
    )
    args = [baseline3] * RPB + [delta_logit_weight] * RPB
    return pl.pallas_call(
        _heights_body,
        grid_spec=grid_spec,
        out_shape=jax.ShapeDtypeStruct((N_REGIONS_OI, N_CLUSTERS, BINWIDTH), jnp.float32),
    )(regions_oi, *args)


HWORDS = N_REGIONS_OI * N_CLUSTERS * BINWIDTH // 2


@functools.lru_cache(maxsize=1)
def _make_gather_kernel():
    mesh = plsc.VectorSubcoreMesh(core_axis_name="c", subcore_axis_name="s")
    HPT = HWORDS // NS

    @functools.partial(
        pl.kernel,
        mesh=mesh,
        out_type=jax.ShapeDtypeStruct((NW, CPW, CHUNK), jnp.float32),
        scratch_types=[
            pltpu.VMEM_SHARED((N_CELLS_PAD,), jnp.int32),
            pltpu.VMEM((CPW, CHUNK), jnp.int32),
            pltpu.VMEM((CPW, CHUNK), jnp.int32),
            pltpu.VMEM((CPW, CHUNK), jnp.int32),
            pltpu.VMEM((CPW, CHUNK), jnp.int32),
            pltpu.VMEM((CPW, CHUNK), jnp.int32),
            pltpu.VMEM((CPW, CHUNK), jnp.float32),
            pltpu.SemaphoreType.DMA,
        ],
    )
    def sc_gather(heights_hbm, labels_hbm, cell_hbm, reg_hbm, coord_hbm, out_hbm,
                  labels_v, cell_v, reg_v, coord_v, cluster_v, flat_v,
                  vals_v, sem):
        sid = lax.axis_index("s")
        wid = sid * NC + lax.axis_index("c")

        with jax.named_scope("ph_in"):
            lbl_slice = pl.ds(sid * (N_CELLS_PAD // NS), N_CELLS_PAD // NS)
            pltpu.sync_copy(labels_hbm.at[lbl_slice], labels_v.at[lbl_slice])
            pltpu.sync_copy(cell_hbm.at[wid], cell_v)
            pltpu.sync_copy(reg_hbm.at[wid], reg_v)
            pltpu.sync_copy(coord_hbm.at[wid], coord_v)
            plsc.subcore_barrier()

        with jax.named_scope("ph_lbl"):
            def lbl_fire(g, carry):
                for u in range(UNROLL):
                    j = g * UNROLL + u
                    pltpu.async_copy(labels_v.at[cell_v.at[j]], cluster_v.at[j], sem)
                return carry

            lax.fori_loop(0, CPW // UNROLL, lbl_fire, 0)

            def lbl_drain(g, carry):
                for u in range(UNROLL):
                    j = g * UNROLL + u
                    pltpu.make_async_copy(
                        labels_v.at[cell_v.at[j]], cluster_v.at[j], sem
                    ).wait()
                return carry

            lax.fori_loop(0, CPW // UNROLL, lbl_drain, 0)

        with jax.named_scope("ph_idx"):
            def idx_body(j, carry):
                for k in range(CHUNK // LANES):
                    sl = pl.ds(k * LANES, LANES)
                    cluster = cluster_v[j, sl]
                    reg = reg_v[j, sl]
                    binix = ((coord_v[j, sl] >> 3) * 41944) >> 20
                    flat_v[j, sl] = (
                        reg * (N_CLUSTERS * BINWIDTH) + cluster * BINWIDTH + binix
                    )
                return carry

            lax.fori_loop(0, CPW, idx_body, 0)

        with jax.named_scope("ph_hgt"):
            def hgt_fire(g, carry):
                for u in range(UNROLL):
                    j = g * UNROLL + u
                    pltpu.async_copy(heights_hbm.at[flat_v.at[j]], vals_v.at[j], sem)
                return carry

            lax.fori_loop(0, CPW // UNROLL, hgt_fire, 0)

            def hgt_drain(g, carry):
                for u in range(UNROLL):
                    j = g * UNROLL + u
                    pltpu.make_async_copy(
                        heights_hbm.at[flat_v.at[j]], vals_v.at[j], sem
                    ).wait()
                return carry

            lax.fori_loop(0, CPW // UNROLL, hgt_drain, 0)

        with jax.named_scope("ph_out"):
            pltpu.sync_copy(vals_v, out_hbm.at[wid])

    return sc_gather


def kernel(baseline_weight, delta_logit_weight, regions_oi, coordinates,
           local_region_ix, local_cell_ix, labels):
    n = coordinates.shape[0]
    roi = regions_oi.astype(jnp.int32)
    heights = _compute_heights(baseline_weight, delta_logit_weight, roi)
    heights_flat = heights.reshape(-1)

    pad = NPAD - n
    shape3 = (NW, CPW, CHUNK)
    cell = jnp.pad(local_cell_ix.astype(jnp.int32), (0, pad)).reshape(shape3)
    reg = jnp.pad(local_region_ix.astype(jnp.int32), (0, pad)).reshape(shape3)
    coord0 = jnp.pad(coordinates[:, 0].astype(jnp.int32), (0, pad)).reshape(shape3)
    labels32 = jnp.pad(labels.astype(jnp.int32), (0, N_CELLS_PAD - labels.shape[0]))

    vals = _make_gather_kernel()(heights_flat, labels32, cell, reg, coord0)
    vals = vals.reshape(-1)[:n]
    return jnp.pad(vals[:, None], ((0, 0), (0, 1)))


# speedup vs baseline: 3.5432x; 1.0086x over previous
import functools
import math

import jax
import jax.numpy as jnp
from jax import lax
from jax.experimental import pallas as pl
from jax.experimental.pallas import tpu as pltpu
from jax.experimental.pallas import tpu_sc as plsc

BINSIZE = 200
BINWIDTH = 500
N_CLUSTERS = 16
N_REGIONS_OI = 256
N_CELLS = 10000
LOG_BINSIZE = math.log(float(BINSIZE))

NC = 2
NS = 16
LANES = 16
NW = NC * NS

CHUNK = 128
CPW = 124
UNROLL = 4
BPW = CHUNK * CPW
NPAD = BPW * NW
N_CELLS_PAD = 10240


RPB = 32


def _heights_body(roi_ref, *refs):
    base_refs = refs[:RPB]
    delta_refs = refs[RPB:2 * RPB]
    out_ref = refs[2 * RPB]
    for k in range(RPB):
        x = base_refs[k][0] + delta_refs[k][0]
        m = jnp.max(x, axis=-1, keepdims=True)
        lse = jnp.log(jnp.sum(jnp.exp(x - m), axis=-1, keepdims=True)) + m
        out_ref[k] = x - lse - LOG_BINSIZE


def _compute_heights(baseline_weight, delta_logit_weight, regions_oi):
    baseline3 = baseline_weight.reshape(baseline_weight.shape[0], 1, BINWIDTH)

    def base_map(k):
        return lambda i, roi: (roi[i * RPB + k], 0, 0)

    grid_spec = pltpu.PrefetchScalarGridSpec(
        num_scalar_prefetch=1,
        grid=(N_REGIONS_OI // RPB,),
        in_specs=(
            [pl.BlockSpec((1, 1, BINWIDTH), base_map(k)) for k in range(RPB)]
            + [pl.BlockSpec((1, N_CLUSTERS, BINWIDTH), base_map(k)) for k in range(RPB)]
        ),
        out_specs=pl.BlockSpec((RPB, N_CLUSTERS, BINWIDTH), lambda i, roi: (i, 0, 0)),
    )
    args = [baseline3] * RPB + [delta_logit_weight] * RPB
    return pl.pallas_call(
        _heights_body,
        grid_spec=grid_spec,
        out_shape=jax.ShapeDtypeStruct((N_REGIONS_OI, N_CLUSTERS, BINWIDTH), jnp.float32),
    )(regions_oi, *args)


HWORDS = N_REGIONS_OI * N_CLUSTERS * BINWIDTH // 2


@functools.lru_cache(maxsize=1)
def _make_gather_kernel():
    mesh = plsc.VectorSubcoreMesh(core_axis_name="c", subcore_axis_name="s")
    HPT = HWORDS // NS

    @functools.partial(
        pl.kernel,
        mesh=mesh,
        out_type=jax.ShapeDtypeStruct((NW, CPW, CHUNK), jnp.float32),
        scratch_types=[
            pltpu.VMEM_SHARED((N_CELLS_PAD,), jnp.int32),
            pltpu.VMEM((CPW, CHUNK), jnp.int32),
            pltpu.VMEM((CPW, CHUNK), jnp.int32),
            pltpu.VMEM((CPW, CHUNK), jnp.int32),
            pltpu.VMEM((CPW, CHUNK), jnp.int32),
            pltpu.VMEM((CPW, CHUNK), jnp.int32),
            pltpu.VMEM((CPW, CHUNK), jnp.float32),
            pltpu.SemaphoreType.DMA,
        ],
    )
    def sc_gather(heights_hbm, labels_hbm, cell_hbm, reg_hbm, coord_hbm, out_hbm,
                  labels_v, cell_v, reg_v, coord_v, cluster_v, flat_v,
                  vals_v, sem):
        sid = lax.axis_index("s")
        wid = sid * NC + lax.axis_index("c")

        with jax.named_scope("ph_in"):
            lbl_slice = pl.ds(sid * (N_CELLS_PAD // NS), N_CELLS_PAD // NS)
            pltpu.sync_copy(labels_hbm.at[lbl_slice], labels_v.at[lbl_slice])
            pltpu.sync_copy(cell_hbm.at[wid], cell_v)
            pltpu.sync_copy(reg_hbm.at[wid], reg_v)
            pltpu.sync_copy(coord_hbm.at[wid], coord_v)
            plsc.subcore_barrier()

        with jax.named_scope("ph_lbl"):
            def lbl_fire(g, carry):
                for u in range(UNROLL):
                    j = g * UNROLL + u
                    pltpu.async_copy(labels_v.at[cell_v.at[j]], cluster_v.at[j], sem)
                return carry

            lax.fori_loop(0, CPW // UNROLL, lbl_fire, 0)

            def lbl_drain(g, carry):
                for u in range(UNROLL):
                    j = g * UNROLL + u
                    pltpu.make_async_copy(
                        labels_v.at[cell_v.at[j]], cluster_v.at[j], sem
                    ).wait()
                return carry

            lax.fori_loop(0, CPW // UNROLL, lbl_drain, 0)

        with jax.named_scope("ph_idx"):
            def idx_body(j, carry):
                for k in range(CHUNK // LANES):
                    sl = pl.ds(k * LANES, LANES)
                    cluster = cluster_v[j, sl]
                    reg = reg_v[j, sl]
                    binix = ((coord_v[j, sl] >> 3) * 41944) >> 20
                    flat_v[j, sl] = (
                        reg * (N_CLUSTERS * BINWIDTH) + cluster * BINWIDTH + binix
                    )
                return carry

            lax.fori_loop(0, CPW, idx_body, 0)

        with jax.named_scope("ph_hgt"):
            def hgt_fire(g, carry):
                for u in range(UNROLL):
                    j = g * UNROLL + u
                    pltpu.async_copy(heights_hbm.at[flat_v.at[j]], vals_v.at[j], sem)
                return carry

            lax.fori_loop(0, CPW // UNROLL, hgt_fire, 0)

            def hgt_drain(g, carry):
                for u in range(UNROLL):
                    j = g * UNROLL + u
                    pltpu.make_async_copy(
                        heights_hbm.at[flat_v.at[j]], vals_v.at[j], sem
                    ).wait()
                return carry

            lax.fori_loop(0, CPW // UNROLL, hgt_drain, 0)

        with jax.named_scope("ph_out"):
            pltpu.sync_copy(vals_v, out_hbm.at[wid])

    return sc_gather


def kernel(baseline_weight, delta_logit_weight, regions_oi, coordinates,
           local_region_ix, local_cell_ix, labels):
    n = coordinates.shape[0]
    roi = regions_oi.astype(jnp.int32)
    heights = _compute_heights(baseline_weight, delta_logit_weight, roi)
    heights_flat = heights.reshape(-1)

    pad = NPAD - n
    shape3 = (NW, CPW, CHUNK)
    cell = jnp.pad(local_cell_ix.astype(jnp.int32), (0, pad)).reshape(shape3)
    reg = jnp.pad(local_region_ix.astype(jnp.int32), (0, pad)).reshape(shape3)
    coord0 = jnp.pad(coordinates[:, 0].astype(jnp.int32), (0, pad)).reshape(shape3)
    labels32 = jnp.pad(labels.astype(jnp.int32), (0, N_CELLS_PAD - labels.shape[0]))

    vals = _make_gather_kernel()(heights_flat, labels32, cell, reg, coord0)
    vals = vals.reshape(-1)[:n]
    return jnp.pad(vals[:, None], ((0, 0), (0, 1)))


# speedup vs baseline: 3.5475x; 1.0012x over previous
import functools
import math

import jax
import jax.numpy as jnp
from jax import lax
from jax.experimental import pallas as pl
from jax.experimental.pallas import tpu as pltpu
from jax.experimental.pallas import tpu_sc as plsc

BINSIZE = 200
BINWIDTH = 500
N_CLUSTERS = 16
N_REGIONS_OI = 256
N_CELLS = 10000
LOG_BINSIZE = math.log(float(BINSIZE))

NC = 2
NS = 16
LANES = 16
NW = NC * NS

CHUNK = 128
CPW = 124
UNROLL = 4
BPW = CHUNK * CPW
NPAD = BPW * NW
N_CELLS_PAD = 10240


RPB = 32


def _heights_body(roi_ref, *refs):
    base_refs = refs[:RPB]
    delta_refs = refs[RPB:2 * RPB]
    out_ref = refs[2 * RPB]
    for k in range(RPB):
        x = base_refs[k][0] + delta_refs[k][0]
        m = jnp.max(x, axis=-1, keepdims=True)
        lse = jnp.log(jnp.sum(jnp.exp(x - m), axis=-1, keepdims=True)) + m
        out_ref[k] = x - lse - LOG_BINSIZE


def _compute_heights(baseline_weight, delta_logit_weight, regions_oi):
    baseline3 = baseline_weight.reshape(baseline_weight.shape[0], 1, BINWIDTH)

    def base_map(k):
        return lambda i, roi: (roi[i * RPB + k], 0, 0)

    grid_spec = pltpu.PrefetchScalarGridSpec(
        num_scalar_prefetch=1,
        grid=(N_REGIONS_OI // RPB,),
        in_specs=(
            [pl.BlockSpec((1, 1, BINWIDTH), base_map(k)) for k in range(RPB)]
            + [pl.BlockSpec((1, N_CLUSTERS, BINWIDTH), base_map(k)) for k in range(RPB)]
        ),
        out_specs=pl.BlockSpec((RPB, N_CLUSTERS, BINWIDTH), lambda i, roi: (i, 0, 0)),
    )
    args = [baseline3] * RPB + [delta_logit_weight] * RPB
    return pl.pallas_call(
        _heights_body,
        grid_spec=grid_spec,
        out_shape=jax.ShapeDtypeStruct((N_REGIONS_OI, N_CLUSTERS, BINWIDTH), jnp.float32),
    )(regions_oi, *args)


@functools.lru_cache(maxsize=1)
def _make_gather_kernel():
    mesh = plsc.VectorSubcoreMesh(core_axis_name="c", subcore_axis_name="s")

    @functools.partial(
        pl.kernel,
        mesh=mesh,
        out_type=jax.ShapeDtypeStruct((NW, CPW, CHUNK), jnp.float32),
        scratch_types=[
            pltpu.VMEM_SHARED((N_CELLS_PAD,), jnp.int32),
            pltpu.VMEM((CPW, CHUNK), jnp.int32),
            pltpu.VMEM((CPW, CHUNK), jnp.int32),
            pltpu.VMEM((CPW, CHUNK), jnp.int32),
            pltpu.VMEM((CPW, CHUNK), jnp.int32),
            pltpu.VMEM((CPW, CHUNK), jnp.int32),
            pltpu.VMEM((CPW, CHUNK), jnp.float32),
            pltpu.SemaphoreType.DMA,
        ],
    )
    def sc_gather(heights_hbm, labels_hbm, cell_hbm, reg_hbm, coord_hbm, out_hbm,
                  labels_v, cell_v, reg_v, coord_v, cluster_v, flat_v,
                  vals_v, sem):
        sid = lax.axis_index("s")
        wid = sid * NC + lax.axis_index("c")

        with jax.named_scope("ph_in"):
            lbl_slice = pl.ds(sid * (N_CELLS_PAD // NS), N_CELLS_PAD // NS)
            pltpu.sync_copy(labels_hbm.at[lbl_slice], labels_v.at[lbl_slice])
            pltpu.sync_copy(cell_hbm.at[wid], cell_v)
            pltpu.sync_copy(reg_hbm.at[wid], reg_v)
            pltpu.sync_copy(coord_hbm.at[wid], coord_v)
            plsc.subcore_barrier()

        with jax.named_scope("ph_lbl"):
            def lbl_fire(g, carry):
                for u in range(UNROLL):
                    j = g * UNROLL + u
                    pltpu.async_copy(labels_v.at[cell_v.at[j]], cluster_v.at[j], sem)
                return carry

            lax.fori_loop(0, CPW // UNROLL, lbl_fire, 0)

            def lbl_drain(g, carry):
                for u in range(UNROLL):
                    j = g * UNROLL + u
                    pltpu.make_async_copy(
                        labels_v.at[cell_v.at[j]], cluster_v.at[j], sem
                    ).wait()
                return carry

            lax.fori_loop(0, CPW // UNROLL, lbl_drain, 0)

        with jax.named_scope("ph_idx"):
            def idx_body(j, carry):
                for k in range(CHUNK // LANES):
                    sl = pl.ds(k * LANES, LANES)
                    cluster = cluster_v[j, sl]
                    reg = reg_v[j, sl]
                    binix = ((coord_v[j, sl] >> 3) * 41944) >> 20
                    flat_v[j, sl] = (
                        reg * (N_CLUSTERS * BINWIDTH) + cluster * BINWIDTH + binix
                    )
                return carry

            lax.fori_loop(0, CPW, idx_body, 0)

        with jax.named_scope("ph_hgt"):
            def hgt_fire(g, carry):
                for u in range(UNROLL):
                    j = g * UNROLL + u
                    pltpu.async_copy(heights_hbm.at[flat_v.at[j]], vals_v.at[j], sem)
                return carry

            lax.fori_loop(0, CPW // UNROLL, hgt_fire, 0)

            def hgt_drain(g, carry):
                for u in range(UNROLL):
                    j = g * UNROLL + u
                    pltpu.make_async_copy(
                        heights_hbm.at[flat_v.at[j]], vals_v.at[j], sem
                    ).wait()
                return carry

            lax.fori_loop(0, CPW // UNROLL, hgt_drain, 0)

        with jax.named_scope("ph_out"):
            pltpu.sync_copy(vals_v, out_hbm.at[wid])

    return sc_gather


def kernel(baseline_weight, delta_logit_weight, regions_oi, coordinates,
           local_region_ix, local_cell_ix, labels):
    n = coordinates.shape[0]
    roi = regions_oi.astype(jnp.int32)
    heights = _compute_heights(baseline_weight, delta_logit_weight, roi)
    heights_flat = heights.reshape(-1)

    pad = NPAD - n
    shape3 = (NW, CPW, CHUNK)
    cell = jnp.pad(local_cell_ix.astype(jnp.int32), (0, pad)).reshape(shape3)
    reg = jnp.pad(local_region_ix.astype(jnp.int32), (0, pad)).reshape(shape3)
    coord0 = jnp.pad(coordinates[:, 0].astype(jnp.int32), (0, pad)).reshape(shape3)
    labels32 = jnp.pad(labels.astype(jnp.int32), (0, N_CELLS_PAD - labels.shape[0]))

    vals = _make_gather_kernel()(heights_flat, labels32, cell, reg, coord0)
    vals = vals.reshape(-1)[:n]
    return jnp.pad(vals[:, None], ((0, 0), (0, 1)))
